# baseline (speedup 1.0000x reference)
import functools

import jax
import jax.numpy as jnp
from jax import lax
from jax.experimental import pallas as pl
from jax.experimental.pallas import tpu as pltpu

N = 512
E = 16384
H = 4
C = 256
HC = H * C
DE = 16
OUT = 200704

BE = 4096
NB = E // BE
BF = 4096
NEG = 0.2
BIGNEG = -1e30


def _f32(x):
    return x.astype(jnp.float32)


def _loop_attr_body(dst_col_ref, efT_ref, deg_ref, loops_ref):
    i = pl.program_id(0)
    dstc = dst_col_ref[...]
    mdst_col = _f32(jnp.broadcast_to(dstc, (BE, N)) ==
                    lax.broadcasted_iota(jnp.int32, (BE, N), 1))

    @pl.when(i == 0)
    def _():
        deg_ref[...] = jnp.zeros_like(deg_ref)
        loops_ref[...] = jnp.zeros_like(loops_ref)

    deg_ref[...] += jnp.sum(mdst_col, axis=0)[None, :]
    loops_ref[...] += jnp.dot(efT_ref[...], mdst_col,
                              preferred_element_type=jnp.float32)

    @pl.when(i == NB - 1)
    def _():
        loops_ref[...] = loops_ref[...] / jnp.maximum(deg_ref[...], 1.0)


def _loop_attrs(dst_col, efT):
    deg, loops = pl.pallas_call(
        _loop_attr_body,
        grid=(NB,),
        in_specs=[
            pl.BlockSpec((BE, 1), lambda i: (i, 0)),
            pl.BlockSpec((DE, BE), lambda i: (0, i)),
        ],
        out_specs=[
            pl.BlockSpec((1, N), lambda i: (0, 0)),
            pl.BlockSpec((DE, N), lambda i: (0, 0)),
        ],
        out_shape=[
            jax.ShapeDtypeStruct((1, N), jnp.float32),
            jax.ShapeDtypeStruct((DE, N), jnp.float32),
        ],
    )(dst_col, efT)
    return loops


def _layer_body(WT_ref, xT_ref, atts_ref, attd_ref, atte_ref, WeT_ref,
                loopsT_ref, b_ref, src_row_ref, dst_row_ref, dst_col_ref,
                efT_ref, out_ref,
                hT_scr, asrc_scr, adst_scr, aself_scr, WaT_scr, amax_scr,
                alpha_scr, B_scr):
    i = pl.program_id(0)

    @pl.when(i == 0)
    def _prep():
        hT = jnp.dot(WT_ref[...], xT_ref[...],
                     preferred_element_type=jnp.float32)
        hT_scr[...] = hT
        ST = _f32(lax.broadcasted_iota(jnp.int32, (H, HC), 0) ==
                  lax.broadcasted_iota(jnp.int32, (H, HC), 1) // C)
        asrcT = jnp.dot(ST * atts_ref[...], hT,
                        preferred_element_type=jnp.float32)
        adstT = jnp.dot(ST * attd_ref[...], hT,
                        preferred_element_type=jnp.float32)
        WaT = jnp.dot(ST * atte_ref[...], WeT_ref[...],
                      preferred_element_type=jnp.float32)
        aloopT = jnp.dot(WaT, loopsT_ref[...],
                         preferred_element_type=jnp.float32)
        sl = asrcT + adstT + aloopT
        aself = jnp.where(sl >= 0, sl, NEG * sl)
        aself_scr[...] = aself
        amax_scr[...] = aself
        asrc_scr[...] = asrcT
        adst_scr[...] = adstT
        WaT_scr[...] = WaT
        B_scr[...] = jnp.zeros_like(B_scr)

    @pl.when(jnp.logical_and(i >= 1, i <= NB))
    def _phase1():
        srcr = src_row_ref[...]
        dstr = dst_row_ref[...]
        io = lax.broadcasted_iota(jnp.int32, (N, BE), 0)
        msrcT = _f32(io == jnp.broadcast_to(srcr, (N, BE)))
        mdstT_b = io == jnp.broadcast_to(dstr, (N, BE))
        a_e = jnp.dot(WaT_scr[...], efT_ref[...],
                      preferred_element_type=jnp.float32)
        a_s = jnp.dot(asrc_scr[...], msrcT,
                      preferred_element_type=jnp.float32)
        a_d = jnp.dot(adst_scr[...], _f32(mdstT_b),
                      preferred_element_type=jnp.float32)
        al = a_s + a_d + a_e
        al = jnp.where(al >= 0, al, NEG * al)
        alpha_scr[:, pl.ds((i - 1) * BE, BE)] = al
        rows = []
        for h in range(H):
            mh = jnp.max(jnp.where(mdstT_b,
                                   jnp.broadcast_to(al[h:h + 1, :], (N, BE)),
                                   BIGNEG), axis=1)
            rows.append(mh[None, :])
        amax_scr[...] = jnp.maximum(amax_scr[...],
                                    jnp.concatenate(rows, axis=0))

    @pl.when(i >= NB + 1)
    def _phase2():
        srcr = src_row_ref[...]
        dstr = dst_row_ref[...]
        dstc = dst_col_ref[...]
        io = lax.broadcasted_iota(jnp.int32, (N, BE), 0)
        msrcT = _f32(io == jnp.broadcast_to(srcr, (N, BE)))
        mdstT = _f32(io == jnp.broadcast_to(dstr, (N, BE)))
        mdst_col = _f32(jnp.broadcast_to(dstc, (BE, N)) ==
                        lax.broadcasted_iota(jnp.int32, (BE, N), 1))
        al = alpha_scr[:, pl.ds((i - NB - 1) * BE, BE)]
        amax_g = jnp.dot(amax_scr[...], mdstT,
                         preferred_element_type=jnp.float32)
        w = jnp.exp(al - amax_g)
        for h in range(H):
            B_scr[h] += jnp.dot(msrcT * w[h:h + 1, :], mdst_col,
                                preferred_element_type=jnp.float32)

    @pl.when(i == 2 * NB)
    def _combine():
        ws = jnp.exp(aself_scr[...] - amax_scr[...])
        hT = hT_scr[...]
        for h in range(H):
            Bh = B_scr[h]
            denom = jnp.sum(Bh, axis=0)[None, :] + ws[h:h + 1, :] + 1e-16
            hTh = hT[h * C:(h + 1) * C, :]
            num = jnp.dot(hTh, Bh, preferred_element_type=jnp.float32) \
                + ws[h:h + 1, :] * hTh
            o = num / denom + b_ref[h * C:(h + 1) * C, :]
            out_ref[h * C:(h + 1) * C, :] = jnp.where(o > 0, o,
                                                      jnp.exp(o) - 1.0)


def _gat_layer(xT, src_row, dst_row, dst_col, efT, loopsT,
               WT, atts, attd, atte, WeT, b_col):
    din = WT.shape[1]
    ebl = lambda i: (0, (i - 1) % NB)
    return pl.pallas_call(
        _layer_body,
        grid=(2 * NB + 1,),
        in_specs=[
            pl.BlockSpec((HC, din), lambda i: (0, 0)),
            pl.BlockSpec((din, N), lambda i: (0, 0)),
            pl.BlockSpec((1, HC), lambda i: (0, 0)),
            pl.BlockSpec((1, HC), lambda i: (0, 0)),
            pl.BlockSpec((1, HC), lambda i: (0, 0)),
            pl.BlockSpec((HC, DE), lambda i: (0, 0)),
            pl.BlockSpec((DE, N), lambda i: (0, 0)),
            pl.BlockSpec((HC, 1), lambda i: (0, 0)),
            pl.BlockSpec((1, BE), ebl),
            pl.BlockSpec((1, BE), ebl),
            pl.BlockSpec((BE, 1), lambda i: ((i - 1) % NB, 0)),
            pl.BlockSpec((DE, BE), ebl),
        ],
        out_specs=pl.BlockSpec((HC, N), lambda i: (0, 0)),
        out_shape=jax.ShapeDtypeStruct((HC, N), jnp.float32),
        scratch_shapes=[
            pltpu.VMEM((HC, N), jnp.float32),
            pltpu.VMEM((H, N), jnp.float32),
            pltpu.VMEM((H, N), jnp.float32),
            pltpu.VMEM((H, N), jnp.float32),
            pltpu.VMEM((H, DE), jnp.float32),
            pltpu.VMEM((H, N), jnp.float32),
            pltpu.VMEM((H, E), jnp.float32),
            pltpu.VMEM((H, N, N), jnp.float32),
        ],
    )(WT, xT, atts, attd, atte, WeT, loopsT, b_col,
      src_row, dst_row, dst_col, efT)


def _final_body(hT_ref, w3_ref, b3_ref, out_ref):
    out_ref[...] = lax.dot_general(
        hT_ref[...], w3_ref[...], (((0,), (0,)), ((), ())),
        preferred_element_type=jnp.float32) + b3_ref[...]


def _final(hT, W3, b3_row):
    nfb = OUT // BF
    return pl.pallas_call(
        _final_body,
        grid=(nfb,),
        in_specs=[
            pl.BlockSpec((HC, N), lambda i: (0, 0)),
            pl.BlockSpec((HC, BF), lambda i: (0, i)),
            pl.BlockSpec((1, BF), lambda i: (0, i)),
        ],
        out_specs=pl.BlockSpec((N, BF), lambda i: (0, i)),
        out_shape=jax.ShapeDtypeStruct((N, OUT), jnp.float32),
        compiler_params=pltpu.CompilerParams(
            dimension_semantics=("parallel",)),
    )(hT, W3, b3_row)


@jax.jit
def kernel(x, edge_index, edge_features, W1, att_src1, att_dst1, We1, att_e1,
           b1, W2, att_src2, att_dst2, We2, att_e2, b2, W3, b3):
    src_row = edge_index[0].reshape(1, E)
    dst_row = edge_index[1].reshape(1, E)
    dst_col = edge_index[1].reshape(E, 1)
    efT = edge_features.T
    xT = x.T

    loopsT = _loop_attrs(dst_col, efT)

    h1T = _gat_layer(xT, src_row, dst_row, dst_col, efT, loopsT,
                     W1.T, att_src1.reshape(1, HC), att_dst1.reshape(1, HC),
                     att_e1.reshape(1, HC), We1.T, b1.reshape(HC, 1))
    h2T = _gat_layer(h1T, src_row, dst_row, dst_col, efT, loopsT,
                     W2.T, att_src2.reshape(1, HC), att_dst2.reshape(1, HC),
                     att_e2.reshape(1, HC), We2.T, b2.reshape(HC, 1))
    return _final(h2T, W3, b3.reshape(1, OUT))


# speedup vs baseline: 1.0077x; 1.0077x over previous
import functools

import jax
import jax.numpy as jnp
from jax import lax
from jax.experimental import pallas as pl
from jax.experimental.pallas import tpu as pltpu

N = 512
E = 16384
H = 4
C = 256
HC = H * C
DE = 16
OUT = 200704

BE = 2048
NB = E // BE
BF = 4096
NEG = 0.2
BIGNEG = -1e30


def _f32(x):
    return x.astype(jnp.float32)


def _loop_attr_body(dst_col_ref, efT_ref, deg_ref, loops_ref):
    i = pl.program_id(0)
    dstc = dst_col_ref[...]
    mdst_col = _f32(jnp.broadcast_to(dstc, (BE, N)) ==
                    lax.broadcasted_iota(jnp.int32, (BE, N), 1))

    @pl.when(i == 0)
    def _():
        deg_ref[...] = jnp.zeros_like(deg_ref)
        loops_ref[...] = jnp.zeros_like(loops_ref)

    deg_ref[...] += jnp.sum(mdst_col, axis=0)[None, :]
    loops_ref[...] += jnp.dot(efT_ref[...], mdst_col,
                              preferred_element_type=jnp.float32)

    @pl.when(i == NB - 1)
    def _():
        loops_ref[...] = loops_ref[...] / jnp.maximum(deg_ref[...], 1.0)


def _loop_attrs(dst_col, efT):
    deg, loops = pl.pallas_call(
        _loop_attr_body,
        grid=(NB,),
        in_specs=[
            pl.BlockSpec((BE, 1), lambda i: (i, 0)),
            pl.BlockSpec((DE, BE), lambda i: (0, i)),
        ],
        out_specs=[
            pl.BlockSpec((1, N), lambda i: (0, 0)),
            pl.BlockSpec((DE, N), lambda i: (0, 0)),
        ],
        out_shape=[
            jax.ShapeDtypeStruct((1, N), jnp.float32),
            jax.ShapeDtypeStruct((DE, N), jnp.float32),
        ],
    )(dst_col, efT)
    return loops


def _layer_body(WT_ref, xT_ref, atts_ref, attd_ref, atte_ref, WeT_ref,
                loopsT_ref, b_ref, efT_ref, src_row_ref, dst_row_ref,
                dst_col_ref, out_ref,
                hT_scr, asrc_scr, adst_scr, aself_scr, amax_scr, ae_scr,
                B_scr):
    i = pl.program_id(0)

    @pl.when(i == 0)
    def _prep():
        hT = jnp.dot(WT_ref[...], xT_ref[...],
                     preferred_element_type=jnp.float32)
        hT_scr[...] = hT
        ST = _f32(lax.broadcasted_iota(jnp.int32, (H, HC), 0) ==
                  lax.broadcasted_iota(jnp.int32, (H, HC), 1) // C)
        asrcT = jnp.dot(ST * atts_ref[...], hT,
                        preferred_element_type=jnp.float32)
        adstT = jnp.dot(ST * attd_ref[...], hT,
                        preferred_element_type=jnp.float32)
        WaT = jnp.dot(ST * atte_ref[...], WeT_ref[...],
                      preferred_element_type=jnp.float32)
        ae_scr[...] = jnp.dot(WaT, efT_ref[...],
                              preferred_element_type=jnp.float32)
        aloopT = jnp.dot(WaT, loopsT_ref[...],
                         preferred_element_type=jnp.float32)
        sl = asrcT + adstT + aloopT
        aself = jnp.where(sl >= 0, sl, NEG * sl)
        aself_scr[...] = aself
        amax_scr[...] = aself
        asrc_scr[...] = asrcT
        adst_scr[...] = adstT
        B_scr[...] = jnp.zeros_like(B_scr)

    @pl.when(i >= 1)
    def _edges():
        srcr = src_row_ref[...]
        dstr = dst_row_ref[...]
        dstc = dst_col_ref[...]
        io = lax.broadcasted_iota(jnp.int32, (N, BE), 0)
        msrcT = _f32(io == jnp.broadcast_to(srcr, (N, BE)))
        mdstT_b = io == jnp.broadcast_to(dstr, (N, BE))
        mdst_col = _f32(jnp.broadcast_to(dstc, (BE, N)) ==
                        lax.broadcasted_iota(jnp.int32, (BE, N), 1))
        a_e = ae_scr[:, pl.ds((i - 1) * BE, BE)]
        a_s = jnp.dot(asrc_scr[...], msrcT,
                      preferred_element_type=jnp.float32)
        a_d = jnp.dot(adst_scr[...], _f32(mdstT_b),
                      preferred_element_type=jnp.float32)
        al = a_s + a_d + a_e
        al = jnp.where(al >= 0, al, NEG * al)
        rows = []
        for h in range(H):
            mh = jnp.max(jnp.where(mdstT_b,
                                   jnp.broadcast_to(al[h:h + 1, :], (N, BE)),
                                   BIGNEG), axis=1)
            rows.append(mh[None, :])
        m_old = amax_scr[...]
        m_new = jnp.maximum(m_old, jnp.concatenate(rows, axis=0))
        amax_scr[...] = m_new
        resc = jnp.exp(m_old - m_new)
        amax_g = jnp.dot(m_new, _f32(mdstT_b),
                         preferred_element_type=jnp.float32)
        w = jnp.exp(al - amax_g)
        for h in range(H):
            B_scr[h] = B_scr[h] * resc[h:h + 1, :] \
                + jnp.dot(msrcT * w[h:h + 1, :], mdst_col,
                          preferred_element_type=jnp.float32)

    @pl.when(i == NB)
    def _combine():
        ws = jnp.exp(aself_scr[...] - amax_scr[...])
        hT = hT_scr[...]
        for h in range(H):
            Bh = B_scr[h]
            denom = jnp.sum(Bh, axis=0)[None, :] + ws[h:h + 1, :] + 1e-16
            hTh = hT[h * C:(h + 1) * C, :]
            num = jnp.dot(hTh, Bh, preferred_element_type=jnp.float32) \
                + ws[h:h + 1, :] * hTh
            o = num / denom + b_ref[h * C:(h + 1) * C, :]
            out_ref[h * C:(h + 1) * C, :] = jnp.where(o > 0, o,
                                                      jnp.exp(o) - 1.0)


def _gat_layer(xT, src_row, dst_row, dst_col, efT, loopsT,
               WT, atts, attd, atte, WeT, b_col):
    din = WT.shape[1]
    ebl = lambda i: (0, (i - 1) % NB)
    return pl.pallas_call(
        _layer_body,
        grid=(NB + 1,),
        in_specs=[
            pl.BlockSpec((HC, din), lambda i: (0, 0)),
            pl.BlockSpec((din, N), lambda i: (0, 0)),
            pl.BlockSpec((1, HC), lambda i: (0, 0)),
            pl.BlockSpec((1, HC), lambda i: (0, 0)),
            pl.BlockSpec((1, HC), lambda i: (0, 0)),
            pl.BlockSpec((HC, DE), lambda i: (0, 0)),
            pl.BlockSpec((DE, N), lambda i: (0, 0)),
            pl.BlockSpec((HC, 1), lambda i: (0, 0)),
            pl.BlockSpec((DE, E), lambda i: (0, 0)),
            pl.BlockSpec((1, BE), ebl),
            pl.BlockSpec((1, BE), ebl),
            pl.BlockSpec((BE, 1), lambda i: ((i - 1) % NB, 0)),
        ],
        out_specs=pl.BlockSpec((HC, N), lambda i: (0, 0)),
        out_shape=jax.ShapeDtypeStruct((HC, N), jnp.float32),
        scratch_shapes=[
            pltpu.VMEM((HC, N), jnp.float32),
            pltpu.VMEM((H, N), jnp.float32),
            pltpu.VMEM((H, N), jnp.float32),
            pltpu.VMEM((H, N), jnp.float32),
            pltpu.VMEM((H, N), jnp.float32),
            pltpu.VMEM((H, E), jnp.float32),
            pltpu.VMEM((H, N, N), jnp.float32),
        ],
    )(WT, xT, atts, attd, atte, WeT, loopsT, b_col, efT,
      src_row, dst_row, dst_col)


def _final_body(hT_ref, w3_ref, b3_ref, out_ref):
    out_ref[...] = lax.dot_general(
        hT_ref[...], w3_ref[...], (((0,), (0,)), ((), ())),
        preferred_element_type=jnp.float32) + b3_ref[...]


def _final(hT, W3, b3_row):
    nfb = OUT // BF
    return pl.pallas_call(
        _final_body,
        grid=(nfb,),
        in_specs=[
            pl.BlockSpec((HC, N), lambda i: (0, 0)),
            pl.BlockSpec((HC, BF), lambda i: (0, i)),
            pl.BlockSpec((1, BF), lambda i: (0, i)),
        ],
        out_specs=pl.BlockSpec((N, BF), lambda i: (0, i)),
        out_shape=jax.ShapeDtypeStruct((N, OUT), jnp.float32),
        compiler_params=pltpu.CompilerParams(
            dimension_semantics=("parallel",)),
    )(hT, W3, b3_row)


@jax.jit
def kernel(x, edge_index, edge_features, W1, att_src1, att_dst1, We1, att_e1,
           b1, W2, att_src2, att_dst2, We2, att_e2, b2, W3, b3):
    src_row = edge_index[0].reshape(1, E)
    dst_row = edge_index[1].reshape(1, E)
    dst_col = edge_index[1].reshape(E, 1)
    efT = edge_features.T
    xT = x.T

    loopsT = _loop_attrs(dst_col, efT)

    h1T = _gat_layer(xT, src_row, dst_row, dst_col, efT, loopsT,
                     W1.T, att_src1.reshape(1, HC), att_dst1.reshape(1, HC),
                     att_e1.reshape(1, HC), We1.T, b1.reshape(HC, 1))
    h2T = _gat_layer(h1T, src_row, dst_row, dst_col, efT, loopsT,
                     W2.T, att_src2.reshape(1, HC), att_dst2.reshape(1, HC),
                     att_e2.reshape(1, HC), We2.T, b2.reshape(HC, 1))
    return _final(h2T, W3, b3.reshape(1, OUT))
